# Initial kernel scaffold; baseline (speedup 1.0000x reference)
import jax
import jax.numpy as jnp
from jax import lax
from jax.experimental import pallas as pl
from jax.experimental.pallas import tpu as pltpu
from jax.experimental.pallas import tpu_sc as plsc

GRAPH_NUM = 3
D = 128
NSAMP = 8192
HALF = NSAMP // 2
LEAKY = 0.2

_NC = 2
_NS = 16
_L = 16
_NW = _NC * _NS
_BPW = NSAMP // _NW


def _leaky(x):
    return jnp.where(x > 0, x, LEAKY * x)


def _sc_gather(fu, uvf, fi, ivf, su, si, n_users, n_items):
    def body(fu_hbm, uvf_hbm, fi_hbm, ivf_hbm, su_hbm, si_hbm,
             fug, uvg, fig, ivg, idx_v, rows_v, sem):
        wid = lax.axis_index("s") * _NC + lax.axis_index("c")
        base = wid * _BPW

        def bump(off):
            for k in range(_BPW // _L):
                sl = pl.ds(k * _L, _L)
                idx_v[sl] = idx_v[sl] + off

        for g in range(GRAPH_NUM):
            out_base = g * NSAMP + base
            pltpu.sync_copy(su_hbm.at[g, pl.ds(base, _BPW)], idx_v)
            pltpu.async_copy(fu_hbm.at[idx_v], rows_v, sem).wait()
            pltpu.sync_copy(rows_v, fug.at[pl.ds(out_base, _BPW)])
            if g:
                bump(g * n_users)
            pltpu.async_copy(uvf_hbm.at[idx_v], rows_v, sem).wait()
            pltpu.sync_copy(rows_v, uvg.at[pl.ds(out_base, _BPW)])
            pltpu.sync_copy(si_hbm.at[g, pl.ds(base, _BPW)], idx_v)
            pltpu.async_copy(fi_hbm.at[idx_v], rows_v, sem).wait()
            pltpu.sync_copy(rows_v, fig.at[pl.ds(out_base, _BPW)])
            if g:
                bump(g * n_items)
            pltpu.async_copy(ivf_hbm.at[idx_v], rows_v, sem).wait()
            pltpu.sync_copy(rows_v, ivg.at[pl.ds(out_base, _BPW)])

    out = jax.ShapeDtypeStruct((GRAPH_NUM * NSAMP, D), jnp.float32)
    kern = pl.kernel(
        body,
        out_type=[out, out, out, out],
        mesh=plsc.VectorSubcoreMesh(core_axis_name="c", subcore_axis_name="s"),
        scratch_types=[
            pltpu.VMEM((_BPW,), jnp.int32),
            pltpu.VMEM((_BPW, D), jnp.float32),
            pltpu.SemaphoreType.DMA,
        ],
    )
    return kern(fu, uvf, fi, ivf, su, si)


def _tc_body(fu_p, fu_n, uv_p, uv_n, fi_p, fi_n, iv_p, iv_n,
             w1, b1, w2, b2, out):
    @pl.when((pl.program_id(0) == 0) & (pl.program_id(1) == 0))
    def _():
        out[...] = jnp.zeros_like(out)

    W1 = w1[...]
    b1v = b1[...]
    w2v = w2[...]
    b2s = b2[0, 0]

    def weight(fu, uv):
        h = (jnp.dot(fu * uv, W1[:D], preferred_element_type=jnp.float32)
             + jnp.dot(fu, W1[D:2 * D], preferred_element_type=jnp.float32)
             + jnp.dot(uv, W1[2 * D:], preferred_element_type=jnp.float32)
             + b1v)
        h = _leaky(h)
        z = jnp.sum(h * w2v, axis=-1) + b2s
        return 1.0 / (1.0 + jnp.exp(-z))

    fu_pv, uv_pv = fu_p[...], uv_p[...]
    fu_nv, uv_nv = fu_n[...], uv_n[...]
    wpos = weight(fu_pv, uv_pv)
    wneg = weight(fu_nv, uv_nv)
    spos = jnp.sum(_leaky(fu_pv * fi_p[...]), axis=-1)
    sneg = jnp.sum(_leaky(fu_nv * fi_n[...]), axis=-1)
    ppos = jnp.sum(_leaky(uv_pv * iv_p[...]), axis=-1)
    pneg = jnp.sum(_leaky(uv_nv * iv_n[...]), axis=-1)
    s = wpos * spos - wneg * sneg
    l = jnp.sum(jnp.maximum(0.0, 1.0 - s * (ppos - pneg)))
    out[...] = out[...] + l


def _tc_loss(fug, uvg, fig, ivg, w1, b1r, w2r, b2r):
    BP = 1024
    nbj = HALF // BP
    nbg = NSAMP // BP

    def pos_map(i, j):
        return (i * nbg + j, 0)

    def neg_map(i, j):
        return (i * nbg + nbj + j, 0)

    rs_p = pl.BlockSpec((BP, D), pos_map)
    rs_n = pl.BlockSpec((BP, D), neg_map)

    def full(shape):
        return pl.BlockSpec(shape, lambda i, j: (0, 0))

    out = pl.pallas_call(
        _tc_body,
        grid=(GRAPH_NUM, nbj),
        in_specs=[rs_p, rs_n, rs_p, rs_n, rs_p, rs_n, rs_p, rs_n,
                  full((3 * D, D)), full((1, D)), full((1, D)), full((1, 1))],
        out_specs=pl.BlockSpec((1, 1), lambda i, j: (0, 0)),
        out_shape=jax.ShapeDtypeStruct((1, 1), jnp.float32),
    )(fug, fug, uvg, uvg, fig, fig, ivg, ivg, w1, b1r, w2r, b2r)
    return out[0, 0]


def kernel(final_user_vector, user_vector, final_item_vector, item_vector,
           suids0, suids1, suids2, siids0, siids1, siids2, W1, b1, W2, b2):
    n_users = final_user_vector.shape[0]
    n_items = final_item_vector.shape[0]
    su = jnp.stack([suids0, suids1, suids2]).astype(jnp.int32)
    si = jnp.stack([siids0, siids1, siids2]).astype(jnp.int32)
    uvf = user_vector.reshape(GRAPH_NUM * n_users, D)
    ivf = item_vector.reshape(GRAPH_NUM * n_items, D)
    fug, uvg, fig, ivg = _sc_gather(
        final_user_vector, uvf, final_item_vector, ivf, su, si,
        n_users, n_items)
    return _tc_loss(fug, uvg, fig, ivg, W1,
                    b1.reshape(1, D), W2.reshape(1, D), b2.reshape(1, 1))


# speedup vs baseline: 5.8621x; 5.8621x over previous
import jax
import jax.numpy as jnp
from jax import lax
from jax.experimental import pallas as pl
from jax.experimental.pallas import tpu as pltpu
from jax.experimental.pallas import tpu_sc as plsc

GRAPH_NUM = 3
D = 128
NSAMP = 8192
HALF = NSAMP // 2
LEAKY = 0.2

_NC = 2
_NS = 16
_L = 16
_NW = _NC * _NS
_BPW = NSAMP // _NW


def _leaky(x):
    return jnp.where(x > 0, x, LEAKY * x)


def _sc_gather(fu, uvf, fi, ivf, su, si, n_users, n_items):
    def body(fu_hbm, uvf_hbm, fi_hbm, ivf_hbm, su_hbm, si_hbm,
             fug, uvg, fig, ivg, idx_v, rows_v, sem):
        wid = lax.axis_index("s") * _NC + lax.axis_index("c")
        base = wid * _BPW

        def bump(off):
            for k in range(_BPW // _L):
                sl = pl.ds(k * _L, _L)
                idx_v[sl] = idx_v[sl] + off

        for g in range(GRAPH_NUM):
            out_base = g * NSAMP + base
            pltpu.sync_copy(su_hbm.at[pl.ds(out_base, _BPW)], idx_v)
            pltpu.async_copy(fu_hbm.at[idx_v], rows_v, sem).wait()
            pltpu.sync_copy(rows_v, fug.at[pl.ds(out_base, _BPW)])
            if g:
                bump(g * n_users)
            pltpu.async_copy(uvf_hbm.at[idx_v], rows_v, sem).wait()
            pltpu.sync_copy(rows_v, uvg.at[pl.ds(out_base, _BPW)])
            pltpu.sync_copy(si_hbm.at[pl.ds(out_base, _BPW)], idx_v)
            pltpu.async_copy(fi_hbm.at[idx_v], rows_v, sem).wait()
            pltpu.sync_copy(rows_v, fig.at[pl.ds(out_base, _BPW)])
            if g:
                bump(g * n_items)
            pltpu.async_copy(ivf_hbm.at[idx_v], rows_v, sem).wait()
            pltpu.sync_copy(rows_v, ivg.at[pl.ds(out_base, _BPW)])

    out = jax.ShapeDtypeStruct((GRAPH_NUM * NSAMP, D), jnp.float32)
    kern = pl.kernel(
        body,
        out_type=[out, out, out, out],
        mesh=plsc.VectorSubcoreMesh(core_axis_name="c", subcore_axis_name="s"),
        scratch_types=[
            pltpu.VMEM((_BPW,), jnp.int32),
            pltpu.VMEM((_BPW, D), jnp.float32),
            pltpu.SemaphoreType.DMA,
        ],
    )
    return kern(fu, uvf, fi, ivf, su, si)


def _tc_body(fu_p, fu_n, uv_p, uv_n, fi_p, fi_n, iv_p, iv_n,
             w1, b1, w2, b2, out):
    @pl.when((pl.program_id(0) == 0) & (pl.program_id(1) == 0))
    def _():
        out[...] = jnp.zeros_like(out)

    W1 = w1[...]
    b1v = b1[...]
    w2v = w2[...]
    b2s = b2[0, 0]

    def weight(fu, uv):
        h = (jnp.dot(fu * uv, W1[:D], preferred_element_type=jnp.float32)
             + jnp.dot(fu, W1[D:2 * D], preferred_element_type=jnp.float32)
             + jnp.dot(uv, W1[2 * D:], preferred_element_type=jnp.float32)
             + b1v)
        h = _leaky(h)
        z = jnp.sum(h * w2v, axis=-1) + b2s
        return 1.0 / (1.0 + jnp.exp(-z))

    fu_pv, uv_pv = fu_p[...], uv_p[...]
    fu_nv, uv_nv = fu_n[...], uv_n[...]
    wpos = weight(fu_pv, uv_pv)
    wneg = weight(fu_nv, uv_nv)
    spos = jnp.sum(_leaky(fu_pv * fi_p[...]), axis=-1)
    sneg = jnp.sum(_leaky(fu_nv * fi_n[...]), axis=-1)
    ppos = jnp.sum(_leaky(uv_pv * iv_p[...]), axis=-1)
    pneg = jnp.sum(_leaky(uv_nv * iv_n[...]), axis=-1)
    s = wpos * spos - wneg * sneg
    l = jnp.sum(jnp.maximum(0.0, 1.0 - s * (ppos - pneg)))
    out[...] = out[...] + l


def _tc_loss(fug, uvg, fig, ivg, w1, b1r, w2r, b2r):
    BP = 1024
    nbj = HALF // BP
    nbg = NSAMP // BP

    def pos_map(i, j):
        return (i * nbg + j, 0)

    def neg_map(i, j):
        return (i * nbg + nbj + j, 0)

    rs_p = pl.BlockSpec((BP, D), pos_map)
    rs_n = pl.BlockSpec((BP, D), neg_map)

    def full(shape):
        return pl.BlockSpec(shape, lambda i, j: (0, 0))

    out = pl.pallas_call(
        _tc_body,
        grid=(GRAPH_NUM, nbj),
        in_specs=[rs_p, rs_n, rs_p, rs_n, rs_p, rs_n, rs_p, rs_n,
                  full((3 * D, D)), full((1, D)), full((1, D)), full((1, 1))],
        out_specs=pl.BlockSpec((1, 1), lambda i, j: (0, 0)),
        out_shape=jax.ShapeDtypeStruct((1, 1), jnp.float32),
    )(fug, fug, uvg, uvg, fig, fig, ivg, ivg, w1, b1r, w2r, b2r)
    return out[0, 0]


def kernel(final_user_vector, user_vector, final_item_vector, item_vector,
           suids0, suids1, suids2, siids0, siids1, siids2, W1, b1, W2, b2):
    n_users = final_user_vector.shape[0]
    n_items = final_item_vector.shape[0]
    su = jnp.concatenate([suids0, suids1, suids2]).astype(jnp.int32)
    si = jnp.concatenate([siids0, siids1, siids2]).astype(jnp.int32)
    uvf = user_vector.reshape(GRAPH_NUM * n_users, D)
    ivf = item_vector.reshape(GRAPH_NUM * n_items, D)
    fug, uvg, fig, ivg = _sc_gather(
        final_user_vector, uvf, final_item_vector, ivf, su, si,
        n_users, n_items)
    return _tc_loss(fug, uvg, fig, ivg, W1,
                    b1.reshape(1, D), W2.reshape(1, D), b2.reshape(1, 1))


# speedup vs baseline: 6.5102x; 1.1105x over previous
import jax
import jax.numpy as jnp
from jax import lax
from jax.experimental import pallas as pl
from jax.experimental.pallas import tpu as pltpu
from jax.experimental.pallas import tpu_sc as plsc

GRAPH_NUM = 3
D = 128
NSAMP = 8192
HALF = NSAMP // 2
LEAKY = 0.2

_NC = 2
_NS = 16
_L = 16
_NW = _NC * _NS
_BPW = NSAMP // _NW


def _leaky(x):
    return jnp.where(x > 0, x, LEAKY * x)


def _sc_gather(fu, uvf, fi, ivf, su, si, n_users, n_items):
    def body(fu_hbm, uvf_hbm, fi_hbm, ivf_hbm, su_hbm, si_hbm,
             fug, uvg, fig, ivg, idx0, idx1, rows0, rows1, sem0, sem1):
        wid = lax.axis_index("s") * _NC + lax.axis_index("c")
        base = wid * _BPW
        idx = (idx0, idx1)
        rows = (rows0, rows1)
        sems = (sem0, sem1)

        rounds = []
        for g in range(GRAPH_NUM):
            ob = g * NSAMP + base
            rounds.append((su_hbm, fu_hbm, 0, fug, ob))
            rounds.append((su_hbm, uvf_hbm, g * n_users, uvg, ob))
            rounds.append((si_hbm, fi_hbm, 0, fig, ob))
            rounds.append((si_hbm, ivf_hbm, g * n_items, ivg, ob))

        def issue(r):
            src, tab, off, _, ob = rounds[r]
            b = r % 2
            pltpu.sync_copy(src.at[pl.ds(ob, _BPW)], idx[b])
            if off:
                for k in range(_BPW // _L):
                    sl = pl.ds(k * _L, _L)
                    idx[b][sl] = idx[b][sl] + off
            return pltpu.async_copy(tab.at[idx[b]], rows[b], sems[b])

        pending = issue(0)
        for r in range(1, len(rounds)):
            nxt = issue(r)
            pending.wait()
            _, _, _, out_ref, ob = rounds[r - 1]
            pltpu.sync_copy(rows[(r - 1) % 2], out_ref.at[pl.ds(ob, _BPW)])
            pending = nxt
        pending.wait()
        _, _, _, out_ref, ob = rounds[-1]
        pltpu.sync_copy(rows[(len(rounds) - 1) % 2],
                        out_ref.at[pl.ds(ob, _BPW)])

    out = jax.ShapeDtypeStruct((GRAPH_NUM * NSAMP, D), jnp.float32)
    kern = pl.kernel(
        body,
        out_type=[out, out, out, out],
        mesh=plsc.VectorSubcoreMesh(core_axis_name="c", subcore_axis_name="s"),
        scratch_types=[
            pltpu.VMEM((_BPW,), jnp.int32),
            pltpu.VMEM((_BPW,), jnp.int32),
            pltpu.VMEM((_BPW, D), jnp.float32),
            pltpu.VMEM((_BPW, D), jnp.float32),
            pltpu.SemaphoreType.DMA,
            pltpu.SemaphoreType.DMA,
        ],
    )
    return kern(fu, uvf, fi, ivf, su, si)


def _tc_body(fu_p, fu_n, uv_p, uv_n, fi_p, fi_n, iv_p, iv_n,
             w1, b1, w2, b2, out):
    @pl.when((pl.program_id(0) == 0) & (pl.program_id(1) == 0))
    def _():
        out[...] = jnp.zeros_like(out)

    W1 = w1[...]
    b1v = b1[...]
    w2v = w2[...]
    b2s = b2[0, 0]

    def weight(fu, uv):
        h = (jnp.dot(fu * uv, W1[:D], preferred_element_type=jnp.float32)
             + jnp.dot(fu, W1[D:2 * D], preferred_element_type=jnp.float32)
             + jnp.dot(uv, W1[2 * D:], preferred_element_type=jnp.float32)
             + b1v)
        h = _leaky(h)
        z = jnp.sum(h * w2v, axis=-1) + b2s
        return 1.0 / (1.0 + jnp.exp(-z))

    fu_pv, uv_pv = fu_p[...], uv_p[...]
    fu_nv, uv_nv = fu_n[...], uv_n[...]
    wpos = weight(fu_pv, uv_pv)
    wneg = weight(fu_nv, uv_nv)
    spos = jnp.sum(_leaky(fu_pv * fi_p[...]), axis=-1)
    sneg = jnp.sum(_leaky(fu_nv * fi_n[...]), axis=-1)
    ppos = jnp.sum(_leaky(uv_pv * iv_p[...]), axis=-1)
    pneg = jnp.sum(_leaky(uv_nv * iv_n[...]), axis=-1)
    s = wpos * spos - wneg * sneg
    l = jnp.sum(jnp.maximum(0.0, 1.0 - s * (ppos - pneg)))
    out[...] = out[...] + l


def _tc_loss(fug, uvg, fig, ivg, w1, b1r, w2r, b2r):
    BP = 1024
    nbj = HALF // BP
    nbg = NSAMP // BP

    def pos_map(i, j):
        return (i * nbg + j, 0)

    def neg_map(i, j):
        return (i * nbg + nbj + j, 0)

    rs_p = pl.BlockSpec((BP, D), pos_map)
    rs_n = pl.BlockSpec((BP, D), neg_map)

    def full(shape):
        return pl.BlockSpec(shape, lambda i, j: (0, 0))

    out = pl.pallas_call(
        _tc_body,
        grid=(GRAPH_NUM, nbj),
        in_specs=[rs_p, rs_n, rs_p, rs_n, rs_p, rs_n, rs_p, rs_n,
                  full((3 * D, D)), full((1, D)), full((1, D)), full((1, 1))],
        out_specs=pl.BlockSpec((1, 1), lambda i, j: (0, 0)),
        out_shape=jax.ShapeDtypeStruct((1, 1), jnp.float32),
    )(fug, fug, uvg, uvg, fig, fig, ivg, ivg, w1, b1r, w2r, b2r)
    return out[0, 0]


def kernel(final_user_vector, user_vector, final_item_vector, item_vector,
           suids0, suids1, suids2, siids0, siids1, siids2, W1, b1, W2, b2):
    n_users = final_user_vector.shape[0]
    n_items = final_item_vector.shape[0]
    su = jnp.concatenate([suids0, suids1, suids2]).astype(jnp.int32)
    si = jnp.concatenate([siids0, siids1, siids2]).astype(jnp.int32)
    uvf = user_vector.reshape(GRAPH_NUM * n_users, D)
    ivf = item_vector.reshape(GRAPH_NUM * n_items, D)
    fug, uvg, fig, ivg = _sc_gather(
        final_user_vector, uvf, final_item_vector, ivf, su, si,
        n_users, n_items)
    return _tc_loss(fug, uvg, fig, ivg, W1,
                    b1.reshape(1, D), W2.reshape(1, D), b2.reshape(1, 1))
